# baseline (speedup 1.0000x reference)
import jax
import jax.numpy as jnp
from jax import lax
from jax.experimental import pallas as pl
from jax.experimental.pallas import tpu as pltpu
from jax.experimental.pallas import tpu_sc as plsc

SEQ_LEN = 200
BATCH = 4096
EMBED_DIM = 64
TOTAL = SEQ_LEN * BATCH

NC = 2
NS = 16
NW = NC * NS

NSPLIT = 4
SPLIT = TOTAL // NSPLIT
PER_W = SPLIT // NW

IDX_ROW = 128
N_IDX_ROWS = PER_W // IDX_ROW
CHUNK = IDX_ROW
NBUF = 8
LAG = 2
LEAD = NBUF - LAG
NCHUNK = PER_W // CHUNK


def _emb_kernel(idx_hbm, table_hbm, out_hbm, idx_v, rows_v, sem_g, sem_o):
    wid = lax.axis_index("s") * NC + lax.axis_index("c")
    base = wid * PER_W

    pltpu.sync_copy(idx_hbm.at[wid], idx_v)

    def fire_gather(c, s):
        pltpu.async_copy(
            table_hbm.at[idx_v.at[c]],
            rows_v.at[s],
            sem_g.at[s],
        )

    def wait_gather(c, s):
        pltpu.make_async_copy(
            table_hbm.at[idx_v.at[c]],
            rows_v.at[s],
            sem_g.at[s],
        ).wait()

    def fire_out(c, s):
        pltpu.async_copy(
            rows_v.at[s],
            out_hbm.at[pl.ds(base + c * CHUNK, CHUNK)],
            sem_o.at[s],
        )

    def wait_out(c, s):
        pltpu.make_async_copy(
            rows_v.at[s],
            out_hbm.at[pl.ds(base + c * CHUNK, CHUNK)],
            sem_o.at[s],
        ).wait()


    for c in range(LEAD):
        fire_gather(c, c % NBUF)
    for c in range(LAG):
        wait_gather(c, c % NBUF)
        fire_out(c, c % NBUF)
        fire_gather(c + LEAD, (c + LEAD) % NBUF)

    STEADY = NCHUNK - LEAD - LAG
    GROUPS = STEADY // NBUF

    def body(t, carry):
        c0 = LAG + t * NBUF
        for i in range(NBUF):
            c = c0 + i
            s = (LAG + i) % NBUF
            wait_gather(c, s)
            fire_out(c, s)
            wait_out(c - LAG, (s - LAG) % NBUF)
            fire_gather(c + LEAD, (s - LAG) % NBUF)
        return carry

    lax.fori_loop(0, GROUPS, body, 0)

    for c in range(LAG + GROUPS * NBUF, NCHUNK - LEAD):
        s = c % NBUF
        wait_gather(c, s)
        fire_out(c, s)
        wait_out(c - LAG, (c - LAG) % NBUF)
        fire_gather(c + LEAD, (c - LAG) % NBUF)

    for c in range(NCHUNK - LEAD, NCHUNK):
        s = c % NBUF
        wait_gather(c, s)
        fire_out(c, s)
    for c in range(NCHUNK - LEAD - LAG, NCHUNK):
        wait_out(c, c % NBUF)


@jax.jit
def kernel(inp, table):
    idx = inp.reshape(NSPLIT, NW, N_IDX_ROWS, IDX_ROW)
    mesh = plsc.VectorSubcoreMesh(core_axis_name="c", subcore_axis_name="s")
    gather = pl.kernel(
        _emb_kernel,
        out_type=jax.ShapeDtypeStruct((SPLIT, EMBED_DIM), jnp.float32),
        mesh=mesh,
        scratch_types=[
            pltpu.VMEM((N_IDX_ROWS, IDX_ROW), jnp.int32),
            pltpu.VMEM((NBUF, CHUNK, EMBED_DIM), jnp.float32),
            pltpu.SemaphoreType.DMA((NBUF,)),
            pltpu.SemaphoreType.DMA((NBUF,)),
        ],
        compiler_params=pltpu.CompilerParams(use_tc_tiling_on_sc=False),
    )
    parts = [gather(idx[k], table) for k in range(NSPLIT)]
    out = jnp.concatenate(parts, axis=0)
    return out.reshape(SEQ_LEN, BATCH, EMBED_DIM)


# speedup vs baseline: 1.1724x; 1.1724x over previous
import jax
import jax.numpy as jnp
from jax import lax
from jax.experimental import pallas as pl
from jax.experimental.pallas import tpu as pltpu
from jax.experimental.pallas import tpu_sc as plsc

SEQ_LEN = 200
BATCH = 4096
EMBED_DIM = 64
TOTAL = SEQ_LEN * BATCH

NC = 2
NS = 16
NW = NC * NS

NSPLIT = 4
SPLIT = TOTAL // NSPLIT
PER_W = SPLIT // NW

IDX_ROW = 128
N_IDX_ROWS = PER_W // IDX_ROW
CHUNK = IDX_ROW
NBUF = 8
LAG = 2
LEAD = NBUF - LAG
NCHUNK = PER_W // CHUNK


def _emb_kernel(idx_hbm, table_hbm, out_hbm, idx_v, rows_v, sem_g, sem_o):
    wid = lax.axis_index("s") * NC + lax.axis_index("c")
    base = wid * PER_W

    pltpu.sync_copy(idx_hbm.at[wid], idx_v)

    def fire_gather(c, s):
        pltpu.async_copy(
            table_hbm.at[idx_v.at[c]],
            rows_v.at[s],
            sem_g.at[s],
        )

    def wait_gather(c, s):
        pltpu.make_async_copy(
            table_hbm.at[idx_v.at[c]],
            rows_v.at[s],
            sem_g.at[s],
        ).wait()

    def fire_out(c, s):
        pltpu.async_copy(
            rows_v.at[s],
            out_hbm.at[pl.ds(base + c * CHUNK, CHUNK)],
            sem_o.at[s],
        )

    def wait_out(c, s):
        pltpu.make_async_copy(
            rows_v.at[s],
            out_hbm.at[pl.ds(base + c * CHUNK, CHUNK)],
            sem_o.at[s],
        ).wait()


    for c in range(LEAD):
        fire_gather(c, c % NBUF)
    for c in range(LAG):
        wait_gather(c, c % NBUF)
        fire_out(c, c % NBUF)
        fire_gather(c + LEAD, (c + LEAD) % NBUF)

    STEADY = NCHUNK - LEAD - LAG
    GROUPS = STEADY // NBUF

    def body(t, carry):
        c0 = LAG + t * NBUF
        for i in range(NBUF):
            c = c0 + i
            s = (LAG + i) % NBUF
            wait_gather(c, s)
            fire_out(c, s)
            wait_out(c - LAG, (s - LAG) % NBUF)
            fire_gather(c + LEAD, (s - LAG) % NBUF)
        return carry

    lax.fori_loop(0, GROUPS, body, 0)

    for c in range(LAG + GROUPS * NBUF, NCHUNK - LEAD):
        s = c % NBUF
        wait_gather(c, s)
        fire_out(c, s)
        wait_out(c - LAG, (c - LAG) % NBUF)
        fire_gather(c + LEAD, (c - LAG) % NBUF)

    for c in range(NCHUNK - LEAD, NCHUNK):
        s = c % NBUF
        wait_gather(c, s)
        fire_out(c, s)
    for c in range(NCHUNK - LEAD - LAG, NCHUNK):
        wait_out(c, c % NBUF)


@jax.jit
def kernel(inp, table):
    idx = inp.reshape(NSPLIT, NW, N_IDX_ROWS, IDX_ROW)
    mesh = plsc.VectorSubcoreMesh(core_axis_name="c", subcore_axis_name="s")
    gather = pl.kernel(
        _emb_kernel,
        out_type=jax.ShapeDtypeStruct((SPLIT, EMBED_DIM), jnp.float32),
        mesh=mesh,
        scratch_types=[
            pltpu.VMEM((N_IDX_ROWS, IDX_ROW), jnp.int32),
            pltpu.VMEM((NBUF, CHUNK, EMBED_DIM), jnp.float32),
            pltpu.SemaphoreType.DMA((NBUF,)),
            pltpu.SemaphoreType.DMA((NBUF,)),
        ],
        compiler_params=pltpu.CompilerParams(use_tc_tiling_on_sc=False),
    )
    parts = [
        gather(idx[k], table).reshape(SEQ_LEN // NSPLIT, BATCH, EMBED_DIM)
        for k in range(NSPLIT)
    ]
    return jnp.concatenate(parts, axis=0)


# speedup vs baseline: 1.2410x; 1.0585x over previous
import jax
import jax.numpy as jnp
from jax import lax
from jax.experimental import pallas as pl
from jax.experimental.pallas import tpu as pltpu
from jax.experimental.pallas import tpu_sc as plsc

SEQ_LEN = 200
BATCH = 4096
EMBED_DIM = 64
TOTAL = SEQ_LEN * BATCH

NC = 2
NS = 16
NW = NC * NS

PER_W = TOTAL // NW

IDX_ROW = 128
N_IDX_ROWS = PER_W // IDX_ROW
CHUNK = IDX_ROW
NBUF = 8
LAG = 2
LEAD = NBUF - LAG
NCHUNK = PER_W // CHUNK


def _emb_kernel(idx_hbm, table_hbm, out_hbm, idx_v, rows_v, sem_g, sem_o):
    wid = lax.axis_index("s") * NC + lax.axis_index("c")
    base = wid * PER_W

    pltpu.sync_copy(idx_hbm.at[wid], idx_v)

    def fire_gather(c, s):
        pltpu.async_copy(
            table_hbm.at[idx_v.at[c]],
            rows_v.at[s],
            sem_g.at[s],
        )

    def wait_gather(c, s):
        pltpu.make_async_copy(
            table_hbm.at[idx_v.at[c]],
            rows_v.at[s],
            sem_g.at[s],
        ).wait()

    def fire_out(c, s):
        pltpu.async_copy(
            rows_v.at[s],
            out_hbm.at[pl.ds(base + c * CHUNK, CHUNK)],
            sem_o.at[s],
        )

    def wait_out(c, s):
        pltpu.make_async_copy(
            rows_v.at[s],
            out_hbm.at[pl.ds(base + c * CHUNK, CHUNK)],
            sem_o.at[s],
        ).wait()


    for c in range(LEAD):
        fire_gather(c, c % NBUF)
    for c in range(LAG):
        wait_gather(c, c % NBUF)
        fire_out(c, c % NBUF)
        fire_gather(c + LEAD, (c + LEAD) % NBUF)

    STEADY = NCHUNK - LEAD - LAG
    GROUPS = STEADY // NBUF

    def body(t, carry):
        c0 = LAG + t * NBUF
        for i in range(NBUF):
            c = c0 + i
            s = (LAG + i) % NBUF
            wait_gather(c, s)
            fire_out(c, s)
            wait_out(c - LAG, (s - LAG) % NBUF)
            fire_gather(c + LEAD, (s - LAG) % NBUF)
        return carry

    lax.fori_loop(0, GROUPS, body, 0)

    for c in range(LAG + GROUPS * NBUF, NCHUNK - LEAD):
        s = c % NBUF
        wait_gather(c, s)
        fire_out(c, s)
        wait_out(c - LAG, (c - LAG) % NBUF)
        fire_gather(c + LEAD, (c - LAG) % NBUF)

    for c in range(NCHUNK - LEAD, NCHUNK):
        s = c % NBUF
        wait_gather(c, s)
        fire_out(c, s)
    for c in range(NCHUNK - LEAD - LAG, NCHUNK):
        wait_out(c, c % NBUF)


@jax.jit
def kernel(inp, table):
    idx = inp.reshape(NW, N_IDX_ROWS, IDX_ROW)
    mesh = plsc.VectorSubcoreMesh(core_axis_name="c", subcore_axis_name="s")
    out = pl.kernel(
        _emb_kernel,
        out_type=jax.ShapeDtypeStruct((TOTAL, EMBED_DIM), jnp.float32),
        mesh=mesh,
        scratch_types=[
            pltpu.VMEM((N_IDX_ROWS, IDX_ROW), jnp.int32),
            pltpu.VMEM((NBUF, CHUNK, EMBED_DIM), jnp.float32),
            pltpu.SemaphoreType.DMA((NBUF,)),
            pltpu.SemaphoreType.DMA((NBUF,)),
        ],
        compiler_params=pltpu.CompilerParams(use_tc_tiling_on_sc=False),
    )(idx, table)
    return out.reshape(SEQ_LEN, BATCH, EMBED_DIM)


# speedup vs baseline: 1.6596x; 1.3373x over previous
import jax
import jax.numpy as jnp
from jax import lax
from jax.experimental import pallas as pl
from jax.experimental.pallas import tpu as pltpu
from jax.experimental.pallas import tpu_sc as plsc

SEQ_LEN = 200
BATCH = 4096
EMBED_DIM = 64
TOTAL = SEQ_LEN * BATCH

NC = 2
NS = 16
NW = NC * NS

PER_W = TOTAL // NW

IDX_ROW = 128
N_IDX_ROWS = PER_W // IDX_ROW
CHUNK = IDX_ROW
NBUF = 8
LAG = 2
LEAD = NBUF - LAG
NCHUNK = PER_W // CHUNK


def _emb_kernel(idx_hbm, table_hbm, out_hbm, idx_v, rows_v, sem_g, sem_o):
    wid = lax.axis_index("s") * NC + lax.axis_index("c")
    base = wid * PER_W

    pltpu.sync_copy(idx_hbm.at[wid], idx_v)

    def fire_gather(c, s):
        pltpu.async_copy(
            table_hbm.at[idx_v.at[c]],
            rows_v.at[s],
            sem_g.at[s],
        )

    def wait_gather(c, s):
        pltpu.make_async_copy(
            table_hbm.at[idx_v.at[c]],
            rows_v.at[s],
            sem_g.at[s],
        ).wait()

    def fire_out(c, s):
        pltpu.async_copy(
            rows_v.at[s],
            out_hbm.at[pl.ds(base + c * CHUNK, CHUNK), pl.ds(0, EMBED_DIM)],
            sem_o.at[s],
        )

    def wait_out(c, s):
        pltpu.make_async_copy(
            rows_v.at[s],
            out_hbm.at[pl.ds(base + c * CHUNK, CHUNK), pl.ds(0, EMBED_DIM)],
            sem_o.at[s],
        ).wait()


    for c in range(LEAD):
        fire_gather(c, c % NBUF)
    for c in range(LAG):
        wait_gather(c, c % NBUF)
        fire_out(c, c % NBUF)
        fire_gather(c + LEAD, (c + LEAD) % NBUF)

    STEADY = NCHUNK - LEAD - LAG
    GROUPS = STEADY // NBUF

    def body(t, carry):
        c0 = LAG + t * NBUF
        for i in range(NBUF):
            c = c0 + i
            s = (LAG + i) % NBUF
            wait_gather(c, s)
            fire_out(c, s)
            wait_out(c - LAG, (s - LAG) % NBUF)
            fire_gather(c + LEAD, (s - LAG) % NBUF)
        return carry

    lax.fori_loop(0, GROUPS, body, 0)

    for c in range(LAG + GROUPS * NBUF, NCHUNK - LEAD):
        s = c % NBUF
        wait_gather(c, s)
        fire_out(c, s)
        wait_out(c - LAG, (c - LAG) % NBUF)
        fire_gather(c + LEAD, (c - LAG) % NBUF)

    for c in range(NCHUNK - LEAD, NCHUNK):
        s = c % NBUF
        wait_gather(c, s)
        fire_out(c, s)
    for c in range(NCHUNK - LEAD - LAG, NCHUNK):
        wait_out(c, c % NBUF)


@jax.jit
def kernel(inp, table):
    idx = inp.reshape(NW, N_IDX_ROWS, IDX_ROW)
    mesh = plsc.VectorSubcoreMesh(core_axis_name="c", subcore_axis_name="s")
    out = pl.kernel(
        _emb_kernel,
        out_type=jax.ShapeDtypeStruct((TOTAL, 2 * EMBED_DIM), jnp.float32),
        mesh=mesh,
        scratch_types=[
            pltpu.VMEM((N_IDX_ROWS, IDX_ROW), jnp.int32),
            pltpu.VMEM((NBUF, CHUNK, EMBED_DIM), jnp.float32),
            pltpu.SemaphoreType.DMA((NBUF,)),
            pltpu.SemaphoreType.DMA((NBUF,)),
        ],
        compiler_params=pltpu.CompilerParams(use_tc_tiling_on_sc=False),
    )(idx, table)
    return out.reshape(SEQ_LEN, BATCH, 2 * EMBED_DIM)[:, :, :EMBED_DIM]


# speedup vs baseline: 1.6919x; 1.0195x over previous
import jax
import jax.numpy as jnp
from jax import lax
from jax.experimental import pallas as pl
from jax.experimental.pallas import tpu as pltpu
from jax.experimental.pallas import tpu_sc as plsc

SEQ_LEN = 200
BATCH = 4096
EMBED_DIM = 64
TOTAL = SEQ_LEN * BATCH

NC = 2
NS = 16
NW = NC * NS

PER_W = TOTAL // NW

IDX_ROW = 128
N_IDX_ROWS = PER_W // IDX_ROW
CHUNK = IDX_ROW
NBUF = 8
LAG = 2
LEAD = NBUF - LAG
NCHUNK = PER_W // CHUNK

VOCAB = 1000000
NTILES = VOCAB // 8
LTB = 16
NLB_TOTAL = NTILES // LTB
LREM_TILES = NTILES - NLB_TOTAL * LTB
LNB = (NLB_TOTAL + NW - 1) // NW


def _launder_kernel(tbl_hbm, out_hbm, buf_v, wide_v, sem_i, sem_o):
    wid = lax.axis_index("s") * NC + lax.axis_index("c")

    def src(g):
        return tbl_hbm.at[pl.ds(g * LTB, LTB)]

    def dst(g):
        return out_hbm.at[pl.ds(g * LTB, LTB)]

    def fire_in(g, s):
        pltpu.async_copy(src(g), buf_v.at[s], sem_i.at[s])

    def wait_in(g, s):
        pltpu.make_async_copy(src(g), buf_v.at[s], sem_i.at[s]).wait()

    def fire_out(g, s):
        pltpu.async_copy(wide_v.at[s], dst(g), sem_o.at[s])

    def wait_out(g, s):
        pltpu.make_async_copy(wide_v.at[s], dst(g), sem_o.at[s]).wait()

    def widen(s):
        def row(r, carry):
            i = r // 8
            j = r % 8
            for k in range(EMBED_DIM // 16):
                wide_v[s, i, j, pl.ds(k * 16, 16)] = (
                    buf_v[s, i, j, pl.ds(k * 16, 16)]
                )
            return carry
        lax.fori_loop(0, LTB * 8, row, 0)

    def batch(b):
        return b * NW + wid

    def guarded(b, fn):
        @pl.when(batch(b) < NLB_TOTAL)
        def _():
            fn(batch(b))

    guarded(0, lambda g: fire_in(g, 0))
    guarded(1, lambda g: fire_in(g, 1))

    def step(b, s):
        def work(g):
            wait_in(g, s)
            widen(s)
            fire_out(g, s)
            wait_out(g, s)
        guarded(b, work)
        guarded(b + 2, lambda g: fire_in(g, s))

    def body(t, carry):
        step(2 * t, 0)
        step(2 * t + 1, 1)
        return carry

    lax.fori_loop(0, LNB // 2, body, 0)
    if LNB % 2:
        step(LNB - 1, (LNB - 1) % 2)

    if LREM_TILES:
        @pl.when(wid == 0)
        def _():
            t0 = NLB_TOTAL * LTB
            pltpu.sync_copy(
                tbl_hbm.at[pl.ds(t0, LREM_TILES)],
                buf_v.at[0, pl.ds(0, LREM_TILES)],
            )
            def row(r, carry):
                i = r // 8
                j = r % 8
                for k in range(EMBED_DIM // 16):
                    wide_v[0, i, j, pl.ds(k * 16, 16)] = (
                        buf_v[0, i, j, pl.ds(k * 16, 16)]
                    )
                return carry
            lax.fori_loop(0, LREM_TILES * 8, row, 0)
            pltpu.sync_copy(
                wide_v.at[0, pl.ds(0, LREM_TILES)],
                out_hbm.at[pl.ds(t0, LREM_TILES)],
            )


def _emb_kernel(idx_hbm, table_hbm, out_hbm, idx_v, rows_v, sem_g, sem_o):
    wid = lax.axis_index("s") * NC + lax.axis_index("c")
    base = wid * PER_W

    pltpu.sync_copy(idx_hbm.at[wid], idx_v)

    def fire_gather(c, s):
        pltpu.async_copy(
            table_hbm.at[idx_v.at[c]],
            rows_v.at[s],
            sem_g.at[s],
        )

    def wait_gather(c, s):
        pltpu.make_async_copy(
            table_hbm.at[idx_v.at[c]],
            rows_v.at[s],
            sem_g.at[s],
        ).wait()

    def fire_out(c, s):
        pltpu.async_copy(
            rows_v.at[s],
            out_hbm.at[pl.ds(base + c * CHUNK, CHUNK), pl.ds(0, EMBED_DIM)],
            sem_o.at[s],
        )

    def wait_out(c, s):
        pltpu.make_async_copy(
            rows_v.at[s],
            out_hbm.at[pl.ds(base + c * CHUNK, CHUNK), pl.ds(0, EMBED_DIM)],
            sem_o.at[s],
        ).wait()


    for c in range(LEAD):
        fire_gather(c, c % NBUF)
    for c in range(LAG):
        wait_gather(c, c % NBUF)
        fire_out(c, c % NBUF)
        fire_gather(c + LEAD, (c + LEAD) % NBUF)

    STEADY = NCHUNK - LEAD - LAG
    GROUPS = STEADY // NBUF

    def body(t, carry):
        c0 = LAG + t * NBUF
        for i in range(NBUF):
            c = c0 + i
            s = (LAG + i) % NBUF
            wait_gather(c, s)
            fire_out(c, s)
            wait_out(c - LAG, (s - LAG) % NBUF)
            fire_gather(c + LEAD, (s - LAG) % NBUF)
        return carry

    lax.fori_loop(0, GROUPS, body, 0)

    for c in range(LAG + GROUPS * NBUF, NCHUNK - LEAD):
        s = c % NBUF
        wait_gather(c, s)
        fire_out(c, s)
        wait_out(c - LAG, (c - LAG) % NBUF)
        fire_gather(c + LEAD, (c - LAG) % NBUF)

    for c in range(NCHUNK - LEAD, NCHUNK):
        s = c % NBUF
        wait_gather(c, s)
        fire_out(c, s)
    for c in range(NCHUNK - LEAD - LAG, NCHUNK):
        wait_out(c, c % NBUF)


@jax.jit
def kernel(inp, table):
    mesh = plsc.VectorSubcoreMesh(core_axis_name="c", subcore_axis_name="s")

    padded = pl.kernel(
        _launder_kernel,
        out_type=jax.ShapeDtypeStruct((NTILES, 8, 2 * EMBED_DIM), jnp.float32),
        mesh=mesh,
        scratch_types=[
            pltpu.VMEM((2, LTB, 8, EMBED_DIM), jnp.float32),
            pltpu.VMEM((2, LTB, 8, 2 * EMBED_DIM), jnp.float32),
            pltpu.SemaphoreType.DMA((2,)),
            pltpu.SemaphoreType.DMA((2,)),
        ],
    )(table.reshape(NTILES, 8, EMBED_DIM))
    flat = padded.reshape(2 * VOCAB, EMBED_DIM)

    idx = (inp * 2).reshape(NW, N_IDX_ROWS, IDX_ROW)
    out = pl.kernel(
        _emb_kernel,
        out_type=jax.ShapeDtypeStruct((TOTAL, 2 * EMBED_DIM), jnp.float32),
        mesh=mesh,
        scratch_types=[
            pltpu.VMEM((N_IDX_ROWS, IDX_ROW), jnp.int32),
            pltpu.VMEM((NBUF, CHUNK, EMBED_DIM), jnp.float32),
            pltpu.SemaphoreType.DMA((NBUF,)),
            pltpu.SemaphoreType.DMA((NBUF,)),
        ],
        compiler_params=pltpu.CompilerParams(use_tc_tiling_on_sc=False),
    )(idx, flat)
    return out.reshape(SEQ_LEN, BATCH, 2 * EMBED_DIM)[:, :, :EMBED_DIM]


# speedup vs baseline: 1.7093x; 1.0103x over previous
import jax
import jax.numpy as jnp
from jax import lax
from jax.experimental import pallas as pl
from jax.experimental.pallas import tpu as pltpu
from jax.experimental.pallas import tpu_sc as plsc

SEQ_LEN = 200
BATCH = 4096
EMBED_DIM = 64
TOTAL = SEQ_LEN * BATCH

NC = 2
NS = 16
NW = NC * NS

PER_W = TOTAL // NW

IDX_ROW = 128
N_IDX_ROWS = PER_W // IDX_ROW
CHUNK = IDX_ROW
NBUF = 8
LAG = 2
LEAD = NBUF - LAG
NCHUNK = PER_W // CHUNK

VOCAB = 1000000
NTILES = VOCAB // 8
LTB = 16
NLB_TOTAL = NTILES // LTB
LREM_TILES = NTILES - NLB_TOTAL * LTB
LNB = (NLB_TOTAL + NW - 1) // NW


def _launder_kernel(tbl_hbm, out_hbm, buf_v, wide_v, sem_i, sem_o):
    wid = lax.axis_index("s") * NC + lax.axis_index("c")

    def src(g):
        return tbl_hbm.at[pl.ds(g * LTB, LTB)]

    def dst(g):
        return out_hbm.at[pl.ds(g * LTB, LTB)]

    def fire_in(g, s):
        pltpu.async_copy(src(g), buf_v.at[s], sem_i.at[s])

    def wait_in(g, s):
        pltpu.make_async_copy(src(g), buf_v.at[s], sem_i.at[s]).wait()

    def fire_out(g, s):
        pltpu.async_copy(wide_v.at[s], dst(g), sem_o.at[s])

    def wait_out(g, s):
        pltpu.make_async_copy(wide_v.at[s], dst(g), sem_o.at[s]).wait()

    def widen(s):
        def tile(i, carry):
            for j in range(8):
                for k in range(EMBED_DIM // 16):
                    wide_v[s, i, j, pl.ds(k * 16, 16)] = (
                        buf_v[s, i, j, pl.ds(k * 16, 16)]
                    )
            return carry
        lax.fori_loop(0, LTB, tile, 0)

    def batch(b):
        return b * NW + wid

    def guarded(b, fn):
        @pl.when(batch(b) < NLB_TOTAL)
        def _():
            fn(batch(b))

    guarded(0, lambda g: fire_in(g, 0))
    guarded(1, lambda g: fire_in(g, 1))

    def step(b, s):
        def work(g):
            wait_in(g, s)
            widen(s)
            fire_out(g, s)
            wait_out(g, s)
        guarded(b, work)
        guarded(b + 2, lambda g: fire_in(g, s))

    def body(t, carry):
        step(2 * t, 0)
        step(2 * t + 1, 1)
        return carry

    lax.fori_loop(0, LNB // 2, body, 0)
    if LNB % 2:
        step(LNB - 1, (LNB - 1) % 2)

    if LREM_TILES:
        @pl.when(wid == 0)
        def _():
            t0 = NLB_TOTAL * LTB
            pltpu.sync_copy(
                tbl_hbm.at[pl.ds(t0, LREM_TILES)],
                buf_v.at[0, pl.ds(0, LREM_TILES)],
            )
            def tile(i, carry):
                for j in range(8):
                    for k in range(EMBED_DIM // 16):
                        wide_v[0, i, j, pl.ds(k * 16, 16)] = (
                            buf_v[0, i, j, pl.ds(k * 16, 16)]
                        )
                return carry
            lax.fori_loop(0, LREM_TILES, tile, 0)
            pltpu.sync_copy(
                wide_v.at[0, pl.ds(0, LREM_TILES)],
                out_hbm.at[pl.ds(t0, LREM_TILES)],
            )


def _emb_kernel(idx_hbm, table_hbm, out_hbm, idx_v, rows_v, sem_g, sem_o):
    wid = lax.axis_index("s") * NC + lax.axis_index("c")
    base = wid * PER_W

    pltpu.sync_copy(idx_hbm.at[wid], idx_v)

    def fire_gather(c, s):
        pltpu.async_copy(
            table_hbm.at[idx_v.at[c]],
            rows_v.at[s],
            sem_g.at[s],
        )

    def wait_gather(c, s):
        pltpu.make_async_copy(
            table_hbm.at[idx_v.at[c]],
            rows_v.at[s],
            sem_g.at[s],
        ).wait()

    def fire_out(c, s):
        pltpu.async_copy(
            rows_v.at[s],
            out_hbm.at[pl.ds(base + c * CHUNK, CHUNK), pl.ds(0, EMBED_DIM)],
            sem_o.at[s],
        )

    def wait_out(c, s):
        pltpu.make_async_copy(
            rows_v.at[s],
            out_hbm.at[pl.ds(base + c * CHUNK, CHUNK), pl.ds(0, EMBED_DIM)],
            sem_o.at[s],
        ).wait()


    for c in range(LEAD):
        fire_gather(c, c % NBUF)
    for c in range(LAG):
        wait_gather(c, c % NBUF)
        fire_out(c, c % NBUF)
        fire_gather(c + LEAD, (c + LEAD) % NBUF)

    STEADY = NCHUNK - LEAD - LAG
    GROUPS = STEADY // NBUF

    def body(t, carry):
        c0 = LAG + t * NBUF
        for i in range(NBUF):
            c = c0 + i
            s = (LAG + i) % NBUF
            wait_gather(c, s)
            fire_out(c, s)
            wait_out(c - LAG, (s - LAG) % NBUF)
            fire_gather(c + LEAD, (s - LAG) % NBUF)
        return carry

    lax.fori_loop(0, GROUPS, body, 0)

    for c in range(LAG + GROUPS * NBUF, NCHUNK - LEAD):
        s = c % NBUF
        wait_gather(c, s)
        fire_out(c, s)
        wait_out(c - LAG, (c - LAG) % NBUF)
        fire_gather(c + LEAD, (c - LAG) % NBUF)

    for c in range(NCHUNK - LEAD, NCHUNK):
        s = c % NBUF
        wait_gather(c, s)
        fire_out(c, s)
    for c in range(NCHUNK - LEAD - LAG, NCHUNK):
        wait_out(c, c % NBUF)


@jax.jit
def kernel(inp, table):
    mesh = plsc.VectorSubcoreMesh(core_axis_name="c", subcore_axis_name="s")

    padded = pl.kernel(
        _launder_kernel,
        out_type=jax.ShapeDtypeStruct((NTILES, 8, 2 * EMBED_DIM), jnp.float32),
        mesh=mesh,
        scratch_types=[
            pltpu.VMEM((2, LTB, 8, EMBED_DIM), jnp.float32),
            pltpu.VMEM((2, LTB, 8, 2 * EMBED_DIM), jnp.float32),
            pltpu.SemaphoreType.DMA((2,)),
            pltpu.SemaphoreType.DMA((2,)),
        ],
    )(table.reshape(NTILES, 8, EMBED_DIM))
    flat = padded.reshape(2 * VOCAB, EMBED_DIM)

    idx = (inp * 2).reshape(NW, N_IDX_ROWS, IDX_ROW)
    out = pl.kernel(
        _emb_kernel,
        out_type=jax.ShapeDtypeStruct((TOTAL, 2 * EMBED_DIM), jnp.float32),
        mesh=mesh,
        scratch_types=[
            pltpu.VMEM((N_IDX_ROWS, IDX_ROW), jnp.int32),
            pltpu.VMEM((NBUF, CHUNK, EMBED_DIM), jnp.float32),
            pltpu.SemaphoreType.DMA((NBUF,)),
            pltpu.SemaphoreType.DMA((NBUF,)),
        ],
        compiler_params=pltpu.CompilerParams(use_tc_tiling_on_sc=False),
    )(idx, flat)
    return out.reshape(SEQ_LEN, BATCH, 2 * EMBED_DIM)[:, :, :EMBED_DIM]
